# Initial kernel scaffold; baseline (speedup 1.0000x reference)
import jax
import jax.numpy as jnp
from jax.experimental import pallas as pl


def kernel(x, edge_index, edge_weight, batch, W0, b0, convW, convB, W1, b1, fcW, fcB, W2, b2):
    raise NotImplementedError("write your pallas kernel here")


# speedup vs baseline: 6.4424x; 6.4424x over previous
import dataclasses
import functools

import jax
import jax.numpy as jnp
from jax import lax
from jax.experimental import pallas as pl
from jax.experimental.pallas import tpu as pltpu
from jax.experimental.pallas import tpu_sc as plsc

_N = 10000
_E = 320000
_F = 128
_G = 64
_NC = 2
_NS = 16
_NW = _NC * _NS
_CH = 128
_NSUB = 80
_EPW = _CH * _NSUB
_EPAD = _EPW * _NW
_RPS = 624
_RTAIL = _N - _RPS * _NS

_BR = 2000
_NB = _N // _BR

_sc_mesh = lambda: plsc.VectorSubcoreMesh(core_axis_name="c", subcore_axis_name="s")


def _sc_params():
    cp = pltpu.CompilerParams()
    if "needs_layout_passes" in pltpu.CompilerParams.__dataclass_fields__:
        cp = dataclasses.replace(cp, needs_layout_passes=False)
    return cp


def _sc_deg(dstR, wR, zrow):
    @functools.partial(
        pl.kernel,
        out_type=jax.ShapeDtypeStruct((_NC, _N, _F), jnp.float32),
        mesh=_sc_mesh(),
        compiler_params=_sc_params(),
        scratch_types=[
            pltpu.VMEM((_NSUPER, _INNER, _CH), jnp.int32),
            pltpu.VMEM((_NSUPER, _INNER, _CH), jnp.float32),
            pltpu.VMEM((_CH, _F), jnp.float32),
            pltpu.VMEM_SHARED((_N, _F), jnp.float32),
        ],
    )
    def k(dst_hbm, w_hbm, z_hbm, out_hbm, dst_v, w_v, row_v, deg_sh):
        cid = lax.axis_index("c")
        sid = lax.axis_index("s")
        wid = cid * _NS + sid

        pltpu.sync_copy(z_hbm.at[pl.ds(0, _RPS)], deg_sh.at[pl.ds(sid * _RPS, _RPS)])

        @pl.when(sid == _NS - 1)
        def _():
            pltpu.sync_copy(z_hbm.at[pl.ds(0, _RTAIL)],
                            deg_sh.at[pl.ds(_RPS * _NS, _RTAIL)])

        pltpu.sync_copy(dst_hbm.at[wid], dst_v)
        pltpu.sync_copy(w_hbm.at[wid], w_v)
        plsc.subcore_barrier()

        @pl.loop(0, _NSUPER)
        def _(t):
            for k_ in range(_INNER):
                w_ref = w_v.at[t, k_]

                @pl.loop(0, _CH)
                def _(e):
                    wv = plsc.load_gather(w_ref, [jnp.full((16,), e, jnp.int32)])
                    for c in range(_F // 16):
                        row_v[e, pl.ds(c * 16, 16)] = wv

                pltpu.sync_copy(row_v, deg_sh.at[dst_v.at[t, k_]], add=True)

        plsc.subcore_barrier()
        pltpu.sync_copy(deg_sh.at[pl.ds(sid * _RPS, _RPS)],
                        out_hbm.at[cid].at[pl.ds(sid * _RPS, _RPS)])

        @pl.when(sid == _NS - 1)
        def _():
            pltpu.sync_copy(deg_sh.at[pl.ds(_RPS * _NS, _RTAIL)],
                            out_hbm.at[cid].at[pl.ds(_RPS * _NS, _RTAIL)])

    return k(dstR, wR, zrow)


_INNER = 8
_NSUPER = _NSUB // _INNER


def _sc_edge(xw, srcR, dstR, wR, zblk):
    @functools.partial(
        pl.kernel,
        out_type=jax.ShapeDtypeStruct((_NC, _N, _F), jnp.float32),
        mesh=_sc_mesh(),
        compiler_params=_sc_params(),
        scratch_types=[
            pltpu.VMEM((2, _INNER, _CH), jnp.int32),
            pltpu.VMEM((2, _INNER, _CH), jnp.int32),
            pltpu.VMEM((2, _INNER, _CH), jnp.float32),
            pltpu.VMEM((2, _CH, _F), jnp.float32),
            pltpu.VMEM_SHARED((_N, _F), jnp.float32),
            pltpu.SemaphoreType.DMA((2,)),
            pltpu.SemaphoreType.DMA((2,)),
            pltpu.SemaphoreType.DMA((2,)),
            pltpu.SemaphoreType.DMA((2,)),
        ],
    )
    def k(xw_hbm, src_hbm, dst_hbm, w_hbm, z_hbm, out_hbm,
          src_v, dst_v, w_v, rows_v, agg_sh, gsem, ssem, dsem, wsem):
        cid = lax.axis_index("c")
        sid = lax.axis_index("s")
        wid = cid * _NS + sid

        pltpu.sync_copy(z_hbm.at[pl.ds(0, _RPS)], agg_sh.at[pl.ds(sid * _RPS, _RPS)])

        @pl.when(sid == _NS - 1)
        def _():
            pltpu.sync_copy(z_hbm.at[pl.ds(0, _RTAIL)],
                            agg_sh.at[pl.ds(_RPS * _NS, _RTAIL)])

        def idx_copies(t, ibuf):
            return (
                pltpu.make_async_copy(src_hbm.at[wid, t], src_v.at[ibuf], ssem.at[ibuf]),
                pltpu.make_async_copy(dst_hbm.at[wid, t], dst_v.at[ibuf], dsem.at[ibuf]),
                pltpu.make_async_copy(w_hbm.at[wid, t], w_v.at[ibuf], wsem.at[ibuf]),
            )

        def g_copy(ibuf, k_, buf):
            return pltpu.make_async_copy(
                xw_hbm.at[src_v.at[ibuf, k_]], rows_v.at[buf], gsem.at[buf])

        def process(ibuf, k_, buf):
            row_buf = rows_v.at[buf]
            w_ref = w_v.at[ibuf, k_]

            @pl.loop(0, _CH)
            def _(e):
                wv = plsc.load_gather(w_ref, [jnp.full((16,), e, jnp.int32)])
                for c in range(_F // 16):
                    sl = pl.ds(c * 16, 16)
                    row_buf[e, sl] = row_buf[e, sl] * wv

            pltpu.sync_copy(row_buf, agg_sh.at[dst_v.at[ibuf, k_]], add=True)

        def super_body(t, ibuf):
            for c in idx_copies(t, ibuf):
                c.wait()

            @pl.when(t + 1 < _NSUPER)
            def _():
                for c in idx_copies(t + 1, 1 - ibuf):
                    c.start()

            g_copy(ibuf, 0, 0).start()
            for k_ in range(_INNER):
                g_copy(ibuf, k_, k_ % 2).wait()
                if k_ + 1 < _INNER:
                    g_copy(ibuf, k_ + 1, (k_ + 1) % 2).start()
                process(ibuf, k_, k_ % 2)

        for c in idx_copies(0, 0):
            c.start()

        plsc.subcore_barrier()

        @pl.loop(0, _NSUPER, step=2)
        def _(t):
            super_body(t, 0)
            super_body(t + 1, 1)

        plsc.subcore_barrier()
        pltpu.sync_copy(agg_sh.at[pl.ds(sid * _RPS, _RPS)],
                        out_hbm.at[cid].at[pl.ds(sid * _RPS, _RPS)])

        @pl.when(sid == _NS - 1)
        def _():
            pltpu.sync_copy(agg_sh.at[pl.ds(_RPS * _NS, _RTAIL)],
                            out_hbm.at[cid].at[pl.ds(_RPS * _NS, _RTAIL)])

    return k(xw, srcR, dstR, wR, zblk)


def _tc_a(x, dis, W0, b0, cw0):
    def body(x_ref, dis_ref, w0_ref, b0_ref, cw_ref, o_ref):
        h = jnp.maximum(
            jnp.dot(x_ref[...], w0_ref[...], preferred_element_type=jnp.float32)
            + b0_ref[...], 0.0)
        o_ref[...] = (jnp.dot(h, cw_ref[...], preferred_element_type=jnp.float32)
                      * dis_ref[...])

    return pl.pallas_call(
        body,
        grid=(_NB,),
        in_specs=[
            pl.BlockSpec((_BR, _F), lambda i: (i, 0)),
            pl.BlockSpec((_BR, 1), lambda i: (i, 0)),
            pl.BlockSpec((_F, _F), lambda i: (0, 0)),
            pl.BlockSpec((1, _F), lambda i: (0, 0)),
            pl.BlockSpec((_F, _F), lambda i: (0, 0)),
        ],
        out_specs=pl.BlockSpec((_BR, _F), lambda i: (i, 0)),
        out_shape=jax.ShapeDtypeStruct((_N, _F), jnp.float32),
    )(x, dis, W0, b0, cw0)


def _tc_b(agg, dis, bias, cw):
    def body(agg_ref, dis_ref, b_ref, w_ref, o_ref):
        dis = dis_ref[...]
        h = jnp.maximum((agg_ref[0] + agg_ref[1]) * dis + b_ref[...], 0.0)
        o_ref[...] = jnp.dot(h, w_ref[...],
                             preferred_element_type=jnp.float32) * dis

    return pl.pallas_call(
        body,
        grid=(_NB,),
        in_specs=[
            pl.BlockSpec((2, _BR, _F), lambda i: (0, i, 0)),
            pl.BlockSpec((_BR, 1), lambda i: (i, 0)),
            pl.BlockSpec((1, _F), lambda i: (0, 0)),
            pl.BlockSpec((_F, _F), lambda i: (0, 0)),
        ],
        out_specs=pl.BlockSpec((_BR, _F), lambda i: (i, 0)),
        out_shape=jax.ShapeDtypeStruct((_N, _F), jnp.float32),
    )(agg, dis, bias, cw)


def _tc_c(agg, dis, bias, batch2d, W1, b1, fcW, fcB, W2, b2):
    def body(agg_ref, dis_ref, b_ref, g_ref, w1_ref, b1_ref, fcw_ref, fcb_ref,
             w2_ref, b2_ref, y_ref, pool_ref, cnt_ref):
        i = pl.program_id(0)

        @pl.when(i == 0)
        def _():
            pool_ref[...] = jnp.zeros_like(pool_ref)
            cnt_ref[...] = jnp.zeros_like(cnt_ref)

        out = jnp.maximum((agg_ref[0] + agg_ref[1]) * dis_ref[...] + b_ref[...], 0.0)
        gids = g_ref[...]
        onehot = (gids == lax.broadcasted_iota(
            jnp.int32, (_BR, _G), 1)).astype(jnp.float32)
        cnt_ref[...] += jnp.sum(onehot, axis=0, keepdims=True)
        lo = gids[0, 0]
        hi = gids[_BR - 1, 0]
        for g in range(_G):
            @pl.when((lo <= g) & (g <= hi))
            def _():
                sel = jnp.where(gids == g, out, 0.0)
                pool_ref[g:g + 1, :] += jnp.sum(sel, axis=0, keepdims=True)

        @pl.when(i == _NB - 1)
        def _():
            cnt_t = jnp.transpose(cnt_ref[...])
            pooled = pool_ref[...] / jnp.maximum(cnt_t, 1.0)
            h = jnp.maximum(
                jnp.dot(pooled, w1_ref[...], preferred_element_type=jnp.float32)
                + b1_ref[...], 0.0)
            h = jnp.maximum(
                jnp.dot(h, fcw_ref[0], preferred_element_type=jnp.float32)
                + fcb_ref[0], 0.0)
            h = jnp.maximum(
                jnp.dot(h, fcw_ref[1], preferred_element_type=jnp.float32)
                + fcb_ref[1], 0.0)
            y_ref[...] = (
                jnp.dot(h, w2_ref[...], preferred_element_type=jnp.float32)
                + b2_ref[...])

    return pl.pallas_call(
        body,
        grid=(_NB,),
        in_specs=[
            pl.BlockSpec((2, _BR, _F), lambda i: (0, i, 0)),
            pl.BlockSpec((_BR, 1), lambda i: (i, 0)),
            pl.BlockSpec((1, _F), lambda i: (0, 0)),
            pl.BlockSpec((_BR, 1), lambda i: (i, 0)),
            pl.BlockSpec((_F, _F), lambda i: (0, 0)),
            pl.BlockSpec((1, _F), lambda i: (0, 0)),
            pl.BlockSpec((2, _F, _F), lambda i: (0, 0, 0)),
            pl.BlockSpec((2, 1, _F), lambda i: (0, 0, 0)),
            pl.BlockSpec((_F, 1), lambda i: (0, 0)),
            pl.BlockSpec((1, 1), lambda i: (0, 0)),
        ],
        out_specs=pl.BlockSpec((_G, 1), lambda i: (0, 0)),
        out_shape=jax.ShapeDtypeStruct((_G, 1), jnp.float32),
        scratch_shapes=[
            pltpu.VMEM((_G, _F), jnp.float32),
            pltpu.VMEM((1, _G), jnp.float32),
        ],
    )(agg, dis, bias, batch2d, W1, b1, fcW, fcB, W2, b2)


def kernel(x, edge_index, edge_weight, batch, W0, b0, convW, convB,
           W1, b1, fcW, fcB, W2, b2):
    pad = _EPAD - _E
    src = jnp.concatenate([edge_index[0], jnp.zeros((pad,), jnp.int32)])
    dst = jnp.concatenate([edge_index[1], jnp.zeros((pad,), jnp.int32)])
    w = jnp.concatenate([edge_weight, jnp.zeros((pad,), jnp.float32)])
    srcR = src.reshape(_NW, _NSUPER, _INNER, _CH)
    dstR = dst.reshape(_NW, _NSUPER, _INNER, _CH)
    wR = w.reshape(_NW, _NSUPER, _INNER, _CH)
    zblk = jnp.zeros((_RPS, _F), jnp.float32)

    dpo = _sc_deg(dstR, wR, zblk)
    deg = dpo[0, :, 0] + dpo[1, :, 0]
    dis = jnp.where(deg > 0, 1.0 / jnp.sqrt(jnp.where(deg > 0, deg, 1.0)),
                    0.0).reshape(_N, 1)
    xw = _tc_a(x, dis, W0, b0.reshape(1, _F), convW[0])
    agg = None
    for i in range(convW.shape[0]):
        agg = _sc_edge(xw, srcR, dstR, wR, zblk)
        if i + 1 < convW.shape[0]:
            xw = _tc_b(agg, dis, convB[i].reshape(1, _F), convW[i + 1])
    y = _tc_c(agg, dis, convB[-1].reshape(1, _F), batch.reshape(_N, 1),
              W1, b1.reshape(1, _F), fcW, fcB.reshape(-1, 1, _F),
              W2, b2.reshape(1, 1))
    return y.reshape(-1)


# speedup vs baseline: 6.5949x; 1.0237x over previous
import dataclasses
import functools

import jax
import jax.numpy as jnp
from jax import lax
from jax.experimental import pallas as pl
from jax.experimental.pallas import tpu as pltpu
from jax.experimental.pallas import tpu_sc as plsc

_N = 10000
_E = 320000
_F = 128
_G = 64
_NC = 2
_NS = 16
_NW = _NC * _NS
_CH = 128
_NSUB = 80
_EPW = _CH * _NSUB
_EPAD = _EPW * _NW
_RPS = 624
_RTAIL = _N - _RPS * _NS

_BR = 2000
_NB = _N // _BR

_sc_mesh = lambda: plsc.VectorSubcoreMesh(core_axis_name="c", subcore_axis_name="s")


def _sc_params():
    cp = pltpu.CompilerParams()
    if "needs_layout_passes" in pltpu.CompilerParams.__dataclass_fields__:
        cp = dataclasses.replace(cp, needs_layout_passes=False)
    return cp


def _sc_deg(dstR, wR, zrow):
    @functools.partial(
        pl.kernel,
        out_type=jax.ShapeDtypeStruct((_NC, _N, _F), jnp.float32),
        mesh=_sc_mesh(),
        compiler_params=_sc_params(),
        scratch_types=[
            pltpu.VMEM((_NSUPER, _INNER, _CH), jnp.int32),
            pltpu.VMEM((_NSUPER, _INNER, _CH), jnp.float32),
            pltpu.VMEM((_CH, _F), jnp.float32),
            pltpu.VMEM_SHARED((_N, _F), jnp.float32),
        ],
    )
    def k(dst_hbm, w_hbm, z_hbm, out_hbm, dst_v, w_v, row_v, deg_sh):
        cid = lax.axis_index("c")
        sid = lax.axis_index("s")
        wid = cid * _NS + sid

        pltpu.sync_copy(z_hbm.at[pl.ds(0, _RPS)], deg_sh.at[pl.ds(sid * _RPS, _RPS)])

        @pl.when(sid == _NS - 1)
        def _():
            pltpu.sync_copy(z_hbm.at[pl.ds(0, _RTAIL)],
                            deg_sh.at[pl.ds(_RPS * _NS, _RTAIL)])

        pltpu.sync_copy(dst_hbm.at[wid], dst_v)
        pltpu.sync_copy(w_hbm.at[wid], w_v)

        zv = jnp.zeros((16,), jnp.float32)

        @pl.loop(0, _CH)
        def _(e):
            for c in range(_F // 16):
                row_v[e, pl.ds(c * 16, 16)] = zv

        plsc.subcore_barrier()

        @pl.loop(0, _NSUPER)
        def _(t):
            for k_ in range(_INNER):
                w_ref = w_v.at[t, k_]

                @pl.loop(0, _CH)
                def _(e):
                    wv = plsc.load_gather(w_ref, [jnp.full((16,), e, jnp.int32)])
                    row_v[e, pl.ds(0, 16)] = wv

                pltpu.sync_copy(row_v, deg_sh.at[dst_v.at[t, k_]], add=True)

        plsc.subcore_barrier()
        pltpu.sync_copy(deg_sh.at[pl.ds(sid * _RPS, _RPS)],
                        out_hbm.at[cid].at[pl.ds(sid * _RPS, _RPS)])

        @pl.when(sid == _NS - 1)
        def _():
            pltpu.sync_copy(deg_sh.at[pl.ds(_RPS * _NS, _RTAIL)],
                            out_hbm.at[cid].at[pl.ds(_RPS * _NS, _RTAIL)])

    return k(dstR, wR, zrow)


_INNER = 8
_NSUPER = _NSUB // _INNER


def _sc_edge(xw, srcR, dstR, wR, zblk):
    @functools.partial(
        pl.kernel,
        out_type=jax.ShapeDtypeStruct((_NC, _N, _F), jnp.float32),
        mesh=_sc_mesh(),
        compiler_params=_sc_params(),
        scratch_types=[
            pltpu.VMEM((2, _INNER, _CH), jnp.int32),
            pltpu.VMEM((2, _INNER, _CH), jnp.int32),
            pltpu.VMEM((2, _INNER, _CH), jnp.float32),
            pltpu.VMEM((2, _CH, _F), jnp.float32),
            pltpu.VMEM_SHARED((_N, _F), jnp.float32),
            pltpu.SemaphoreType.DMA((2,)),
            pltpu.SemaphoreType.DMA((2,)),
            pltpu.SemaphoreType.DMA((2,)),
            pltpu.SemaphoreType.DMA((2,)),
        ],
    )
    def k(xw_hbm, src_hbm, dst_hbm, w_hbm, z_hbm, out_hbm,
          src_v, dst_v, w_v, rows_v, agg_sh, gsem, ssem, dsem, wsem):
        cid = lax.axis_index("c")
        sid = lax.axis_index("s")
        wid = cid * _NS + sid

        pltpu.sync_copy(z_hbm.at[pl.ds(0, _RPS)], agg_sh.at[pl.ds(sid * _RPS, _RPS)])

        @pl.when(sid == _NS - 1)
        def _():
            pltpu.sync_copy(z_hbm.at[pl.ds(0, _RTAIL)],
                            agg_sh.at[pl.ds(_RPS * _NS, _RTAIL)])

        def idx_copies(t, ibuf):
            return (
                pltpu.make_async_copy(src_hbm.at[wid, t], src_v.at[ibuf], ssem.at[ibuf]),
                pltpu.make_async_copy(dst_hbm.at[wid, t], dst_v.at[ibuf], dsem.at[ibuf]),
                pltpu.make_async_copy(w_hbm.at[wid, t], w_v.at[ibuf], wsem.at[ibuf]),
            )

        def g_copy(ibuf, k_, buf):
            return pltpu.make_async_copy(
                xw_hbm.at[src_v.at[ibuf, k_]], rows_v.at[buf], gsem.at[buf])

        def process(ibuf, k_, buf):
            row_buf = rows_v.at[buf]
            w_ref = w_v.at[ibuf, k_]

            @pl.loop(0, _CH, step=2)
            def _(e):
                for u in range(2):
                    eu = e + u
                    wv = plsc.load_gather(w_ref, [jnp.full((16,), eu, jnp.int32)])
                    for c in range(_F // 16):
                        sl = pl.ds(c * 16, 16)
                        row_buf[eu, sl] = row_buf[eu, sl] * wv

            pltpu.sync_copy(row_buf, agg_sh.at[dst_v.at[ibuf, k_]], add=True)

        def super_body(t, ibuf):
            for c in idx_copies(t, ibuf):
                c.wait()

            @pl.when(t + 1 < _NSUPER)
            def _():
                for c in idx_copies(t + 1, 1 - ibuf):
                    c.start()

            g_copy(ibuf, 0, 0).start()
            for k_ in range(_INNER):
                g_copy(ibuf, k_, k_ % 2).wait()
                if k_ + 1 < _INNER:
                    g_copy(ibuf, k_ + 1, (k_ + 1) % 2).start()
                process(ibuf, k_, k_ % 2)

        for c in idx_copies(0, 0):
            c.start()

        plsc.subcore_barrier()

        @pl.loop(0, _NSUPER, step=2)
        def _(t):
            super_body(t, 0)
            super_body(t + 1, 1)

        plsc.subcore_barrier()
        pltpu.sync_copy(agg_sh.at[pl.ds(sid * _RPS, _RPS)],
                        out_hbm.at[cid].at[pl.ds(sid * _RPS, _RPS)])

        @pl.when(sid == _NS - 1)
        def _():
            pltpu.sync_copy(agg_sh.at[pl.ds(_RPS * _NS, _RTAIL)],
                            out_hbm.at[cid].at[pl.ds(_RPS * _NS, _RTAIL)])

    return k(xw, srcR, dstR, wR, zblk)


def _tc_a(x, dis, W0, b0, cw0):
    def body(x_ref, dis_ref, w0_ref, b0_ref, cw_ref, o_ref):
        h = jnp.maximum(
            jnp.dot(x_ref[...], w0_ref[...], preferred_element_type=jnp.float32)
            + b0_ref[...], 0.0)
        o_ref[...] = (jnp.dot(h, cw_ref[...], preferred_element_type=jnp.float32)
                      * dis_ref[...])

    return pl.pallas_call(
        body,
        grid=(_NB,),
        in_specs=[
            pl.BlockSpec((_BR, _F), lambda i: (i, 0)),
            pl.BlockSpec((_BR, 1), lambda i: (i, 0)),
            pl.BlockSpec((_F, _F), lambda i: (0, 0)),
            pl.BlockSpec((1, _F), lambda i: (0, 0)),
            pl.BlockSpec((_F, _F), lambda i: (0, 0)),
        ],
        out_specs=pl.BlockSpec((_BR, _F), lambda i: (i, 0)),
        out_shape=jax.ShapeDtypeStruct((_N, _F), jnp.float32),
    )(x, dis, W0, b0, cw0)


def _tc_b(agg, dis, bias, cw):
    def body(agg_ref, dis_ref, b_ref, w_ref, o_ref):
        dis = dis_ref[...]
        h = jnp.maximum((agg_ref[0] + agg_ref[1]) * dis + b_ref[...], 0.0)
        o_ref[...] = jnp.dot(h, w_ref[...],
                             preferred_element_type=jnp.float32) * dis

    return pl.pallas_call(
        body,
        grid=(_NB,),
        in_specs=[
            pl.BlockSpec((2, _BR, _F), lambda i: (0, i, 0)),
            pl.BlockSpec((_BR, 1), lambda i: (i, 0)),
            pl.BlockSpec((1, _F), lambda i: (0, 0)),
            pl.BlockSpec((_F, _F), lambda i: (0, 0)),
        ],
        out_specs=pl.BlockSpec((_BR, _F), lambda i: (i, 0)),
        out_shape=jax.ShapeDtypeStruct((_N, _F), jnp.float32),
    )(agg, dis, bias, cw)


def _tc_c(agg, dis, bias, batch2d, W1, b1, fcW, fcB, W2, b2):
    def body(agg_ref, dis_ref, b_ref, g_ref, w1_ref, b1_ref, fcw_ref, fcb_ref,
             w2_ref, b2_ref, y_ref, pool_ref, cnt_ref):
        i = pl.program_id(0)

        @pl.when(i == 0)
        def _():
            pool_ref[...] = jnp.zeros_like(pool_ref)
            cnt_ref[...] = jnp.zeros_like(cnt_ref)

        out = jnp.maximum((agg_ref[0] + agg_ref[1]) * dis_ref[...] + b_ref[...], 0.0)
        gids = g_ref[...]
        onehot = (gids == lax.broadcasted_iota(
            jnp.int32, (_BR, _G), 1)).astype(jnp.float32)
        cnt_ref[...] += jnp.sum(onehot, axis=0, keepdims=True)
        lo = gids[0, 0]
        hi = gids[_BR - 1, 0]
        for g in range(_G):
            @pl.when((lo <= g) & (g <= hi))
            def _():
                sel = jnp.where(gids == g, out, 0.0)
                pool_ref[g:g + 1, :] += jnp.sum(sel, axis=0, keepdims=True)

        @pl.when(i == _NB - 1)
        def _():
            cnt_t = jnp.transpose(cnt_ref[...])
            pooled = pool_ref[...] / jnp.maximum(cnt_t, 1.0)
            h = jnp.maximum(
                jnp.dot(pooled, w1_ref[...], preferred_element_type=jnp.float32)
                + b1_ref[...], 0.0)
            h = jnp.maximum(
                jnp.dot(h, fcw_ref[0], preferred_element_type=jnp.float32)
                + fcb_ref[0], 0.0)
            h = jnp.maximum(
                jnp.dot(h, fcw_ref[1], preferred_element_type=jnp.float32)
                + fcb_ref[1], 0.0)
            y_ref[...] = (
                jnp.dot(h, w2_ref[...], preferred_element_type=jnp.float32)
                + b2_ref[...])

    return pl.pallas_call(
        body,
        grid=(_NB,),
        in_specs=[
            pl.BlockSpec((2, _BR, _F), lambda i: (0, i, 0)),
            pl.BlockSpec((_BR, 1), lambda i: (i, 0)),
            pl.BlockSpec((1, _F), lambda i: (0, 0)),
            pl.BlockSpec((_BR, 1), lambda i: (i, 0)),
            pl.BlockSpec((_F, _F), lambda i: (0, 0)),
            pl.BlockSpec((1, _F), lambda i: (0, 0)),
            pl.BlockSpec((2, _F, _F), lambda i: (0, 0, 0)),
            pl.BlockSpec((2, 1, _F), lambda i: (0, 0, 0)),
            pl.BlockSpec((_F, 1), lambda i: (0, 0)),
            pl.BlockSpec((1, 1), lambda i: (0, 0)),
        ],
        out_specs=pl.BlockSpec((_G, 1), lambda i: (0, 0)),
        out_shape=jax.ShapeDtypeStruct((_G, 1), jnp.float32),
        scratch_shapes=[
            pltpu.VMEM((_G, _F), jnp.float32),
            pltpu.VMEM((1, _G), jnp.float32),
        ],
    )(agg, dis, bias, batch2d, W1, b1, fcW, fcB, W2, b2)


def kernel(x, edge_index, edge_weight, batch, W0, b0, convW, convB,
           W1, b1, fcW, fcB, W2, b2):
    pad = _EPAD - _E
    src = jnp.concatenate([edge_index[0], jnp.zeros((pad,), jnp.int32)])
    dst = jnp.concatenate([edge_index[1], jnp.zeros((pad,), jnp.int32)])
    w = jnp.concatenate([edge_weight, jnp.zeros((pad,), jnp.float32)])
    srcR = src.reshape(_NW, _NSUPER, _INNER, _CH)
    dstR = dst.reshape(_NW, _NSUPER, _INNER, _CH)
    wR = w.reshape(_NW, _NSUPER, _INNER, _CH)
    zblk = jnp.zeros((_RPS, _F), jnp.float32)

    dpo = _sc_deg(dstR, wR, zblk)
    deg = dpo[0, :, 0] + dpo[1, :, 0]
    dis = jnp.where(deg > 0, 1.0 / jnp.sqrt(jnp.where(deg > 0, deg, 1.0)),
                    0.0).reshape(_N, 1)
    xw = _tc_a(x, dis, W0, b0.reshape(1, _F), convW[0])
    agg = None
    for i in range(convW.shape[0]):
        agg = _sc_edge(xw, srcR, dstR, wR, zblk)
        if i + 1 < convW.shape[0]:
            xw = _tc_b(agg, dis, convB[i].reshape(1, _F), convW[i + 1])
    y = _tc_c(agg, dis, convB[-1].reshape(1, _F), batch.reshape(_N, 1),
              W1, b1.reshape(1, _F), fcW, fcB.reshape(-1, 1, _F),
              W2, b2.reshape(1, 1))
    return y.reshape(-1)
